# baseline (speedup 1.0000x reference)
import jax
import jax.numpy as jnp
from jax.experimental import pallas as pl
from jax.experimental.pallas import tpu as pltpu

_ALPHA = 0.3
_INV2A2 = 1.0 / (2.0 * _ALPHA * _ALPHA)


def _chamfer_kernel(x_ref, y_ref, y2_ref, out_ref, cmin_ref, acc_ref):
    b = pl.program_id(0)
    i = pl.program_id(1)
    nb = pl.num_programs(0)
    ni = pl.num_programs(1)

    @pl.when((b == 0) & (i == 0))
    def _init_acc():
        acc_ref[0, 0] = 0.0

    @pl.when(i == 0)
    def _init_cmin():
        cmin_ref[...] = jnp.full_like(cmin_ref, jnp.inf)

    x = x_ref[0]
    y = y_ref[0]
    xy = jax.lax.dot_general(x, y, (((1,), (1,)), ((), ())),
                             preferred_element_type=jnp.float32)
    x2 = jnp.sum(x * x, axis=1, keepdims=True)
    d = (x2 + y2_ref[0]) + xy

    rmin = jnp.min(d, axis=1)
    acc_ref[0, 0] += jnp.sum(jnp.exp(-(rmin * rmin) * _INV2A2) * rmin)

    cmin_ref[...] = jnp.minimum(cmin_ref[...], jnp.min(d, axis=0, keepdims=True))

    @pl.when(i == ni - 1)
    def _batch_end():
        c = cmin_ref[0]
        acc_ref[0, 0] += jnp.sum(jnp.exp(-(c * c) * _INV2A2) * c)

    @pl.when((b == nb - 1) & (i == ni - 1))
    def _final():
        out_ref[0, 0] = acc_ref[0, 0] / nb


def _chamfer_pallas(xp, yp, y2):
    B, N, _ = xp.shape
    M = yp.shape[1]
    TI = 2048
    ni = N // TI
    return pl.pallas_call(
        _chamfer_kernel,
        grid=(B, ni),
        in_specs=[
            pl.BlockSpec((1, TI, 3), lambda b, i: (b, i, 0)),
            pl.BlockSpec((1, M, 3), lambda b, i: (b, 0, 0)),
            pl.BlockSpec((1, 1, M), lambda b, i: (b, 0, 0)),
        ],
        out_specs=pl.BlockSpec(memory_space=pltpu.SMEM),
        out_shape=jax.ShapeDtypeStruct((1, 1), jnp.float32),
        scratch_shapes=[
            pltpu.VMEM((1, M), jnp.float32),
            pltpu.SMEM((1, 1), jnp.float32),
        ],
    )(xp, yp, y2)


def kernel(verts, target_points, target_normals):
    t = target_points
    v = verts
    y2 = jnp.sum(v * v, axis=-1)[:, None, :]
    out = _chamfer_pallas(t, -2.0 * v, y2)
    return out[0, 0]


# speedup vs baseline: 1.0555x; 1.0555x over previous
import jax
import jax.numpy as jnp
from jax.experimental import pallas as pl
from jax.experimental.pallas import tpu as pltpu

_ALPHA = 0.3
_INV2A2 = 1.0 / (2.0 * _ALPHA * _ALPHA)


def _chamfer_kernel(x_ref, y_ref, y2_ref, out_ref, cmin_ref, acc_ref):
    b = pl.program_id(0)
    i = pl.program_id(1)
    nb = pl.num_programs(0)
    ni = pl.num_programs(1)

    @pl.when((b == 0) & (i == 0))
    def _init_acc():
        acc_ref[0, 0] = 0.0

    @pl.when(i == 0)
    def _init_cmin():
        cmin_ref[...] = jnp.full_like(cmin_ref, jnp.inf)

    x = x_ref[0]
    y = y_ref[0]
    xy = jax.lax.dot_general(-2.0 * x, y, (((1,), (1,)), ((), ())),
                             preferred_element_type=jnp.float32)
    x2 = jnp.sum(x * x, axis=1, keepdims=True)
    d = (x2 + y2_ref[0]) + xy

    rmin = jnp.min(d, axis=1)
    acc_ref[0, 0] += jnp.sum(jnp.exp(-(rmin * rmin) * _INV2A2) * rmin)

    cmin_ref[...] = jnp.minimum(cmin_ref[...], jnp.min(d, axis=0, keepdims=True))

    @pl.when(i == ni - 1)
    def _batch_end():
        c = cmin_ref[0]
        acc_ref[0, 0] += jnp.sum(jnp.exp(-(c * c) * _INV2A2) * c)

    @pl.when((b == nb - 1) & (i == ni - 1))
    def _final():
        out_ref[0, 0] = acc_ref[0, 0] / nb


def _chamfer_pallas(xp, yp, y2):
    B, N, _ = xp.shape
    M = yp.shape[1]
    TI = 2048
    ni = N // TI
    return pl.pallas_call(
        _chamfer_kernel,
        grid=(B, ni),
        in_specs=[
            pl.BlockSpec((1, TI, 3), lambda b, i: (b, i, 0)),
            pl.BlockSpec((1, M, 3), lambda b, i: (b, 0, 0)),
            pl.BlockSpec((1, 1, M), lambda b, i: (b, 0, 0)),
        ],
        out_specs=pl.BlockSpec(memory_space=pltpu.SMEM),
        out_shape=jax.ShapeDtypeStruct((1, 1), jnp.float32),
        scratch_shapes=[
            pltpu.VMEM((1, M), jnp.float32),
            pltpu.SMEM((1, 1), jnp.float32),
        ],
    )(xp, yp, y2)


def kernel(verts, target_points, target_normals):
    t = target_points
    v = verts
    y2 = jnp.sum(v * v, axis=-1)[:, None, :]
    out = _chamfer_pallas(t, v, y2)
    return out[0, 0]
